# baseline (speedup 1.0000x reference)
import functools

import jax
import jax.numpy as jnp
from jax import lax
from jax.experimental import pallas as pl
from jax.experimental.pallas import tpu as pltpu
from jax.experimental.pallas import tpu_sc as plsc

F = 128
CHUNK = 16
NROW = 4
NIDX = 8


def _sc_segment_sum(table, idx3, zrows, width, cpt):
    nrow = table.shape[0]
    n = nrow - 1
    info = plsc.get_sparse_core_info()
    ncores, nsub = info.num_cores, info.num_subcores
    n_pad = zrows.shape[0]
    rows_per_tile = n_pad // nsub
    assert n_pad >= nrow and rows_per_tile % 8 == 0 and (cpt - 2) % (2 * NROW) == 0

    mesh = plsc.VectorSubcoreMesh(core_axis_name="c", subcore_axis_name="s")

    @functools.partial(
        pl.kernel,
        mesh=mesh,
        out_type=jax.ShapeDtypeStruct((ncores, n_pad, width), jnp.float32),
        scratch_types=(
            [pltpu.VMEM((2, CHUNK), jnp.int32) for _ in range(NIDX)]
            + [pltpu.VMEM((CHUNK, width), jnp.float32) for _ in range(NROW)]
            + [pltpu.VMEM_SHARED((n_pad, width), jnp.float32)]
            + [pltpu.SemaphoreType.DMA for _ in range(NIDX + 2 * NROW + 1)]
        ),
        compiler_params=pltpu.CompilerParams(use_tc_tiling_on_sc=False),
    )
    def agg(table_hbm, idx_hbm, z_hbm, out_hbm, *rest):
        idxb = rest[:NIDX]
        rows = rest[NIDX:NIDX + NROW]
        acc = rest[NIDX + NROW]
        sems = rest[NIDX + NROW + 1:]
        isem = sems[:NIDX]
        gsem = sems[NIDX:NIDX + NROW]
        ssem = sems[NIDX + NROW:NIDX + 2 * NROW]
        zsem = sems[NIDX + 2 * NROW]

        c = lax.axis_index("c")
        s = lax.axis_index("s")
        tile = c * nsub + s
        row0 = s * rows_per_tile

        zero_cp = pltpu.make_async_copy(
            z_hbm.at[pl.ds(row0, rows_per_tile)],
            acc.at[pl.ds(row0, rows_per_tile)], zsem)
        zero_cp.start()

        def idx_load(i, q):
            return pltpu.make_async_copy(idx_hbm.at[tile, i], idxb[q], isem[q])

        def gather(q, r):
            return pltpu.make_async_copy(
                table_hbm.at[idxb[q].at[0]], rows[r], gsem[r])

        class _Scatter:
            def __init__(self, q, r):
                self.q, self.r = q, r

            def start(self):
                pltpu.async_copy(rows[self.r], acc.at[idxb[self.q].at[1]],
                                 ssem[self.r], add=True)

            def wait(self):
                pltpu.make_async_copy(rows[self.r],
                                      acc.at[idxb[self.q].at[1]],
                                      ssem[self.r]).wait()

        def scatter(q, r):
            return _Scatter(q, r)

        for j in range(5):
            idx_load(j, j).start()
        for j in range(2):
            idx_load(j, j).wait()
            gather(j, j).start()

        zero_cp.wait()
        plsc.subcore_barrier()

        for i in range(2):
            idx_load(i + 2, i + 2).wait()
            gather(i + 2, (i + 2) % NROW).start()
            idx_load(i + 5, (i + 5) % NIDX).start()
            gather(i, i).wait()
            scatter(i, i).start()

        def rnd(rr, _):
            i0 = rr * (2 * NROW) + 2
            for b in range(2 * NROW):
                i = i0 + b
                scatter(0, b % NROW).wait()
                idx_load(i + 2, (4 + b) % NIDX).wait()
                gather((4 + b) % NIDX, b % NROW).start()
                idx_load(i + 5, (7 + b) % NIDX).start()
                gather(0, (2 + b) % NROW).wait()
                scatter((2 + b) % NIDX, (2 + b) % NROW).start()
            return 0
        lax.fori_loop(0, (cpt - 2) // (2 * NROW), rnd, 0)

        for i in range(cpt - 2, cpt):
            scatter(0, i % NROW).wait()
        for i in range(cpt, cpt + 2):
            gather(0, i % NROW).wait()
        for i in range(cpt + 2, cpt + 5):
            idx_load(i, i % NIDX).wait()
        plsc.subcore_barrier()

        pltpu.sync_copy(acc.at[pl.ds(row0, rows_per_tile)],
                        out_hbm.at[c, pl.ds(row0, rows_per_tile)])

    return agg(table, idx3, zrows)


def _tc_layer1(x, p0, p1, w_self, w_neigh, b):
    n = x.shape[0]
    w1 = p0.shape[1]
    rblk = 1000
    grid = (n // rblk,)

    def body(x_ref, a0_ref, a1_ref, ws_ref, wn_ref, b_ref, o_ref, invd_ref):
        xv = x_ref[...]
        a0 = a0_ref[...]
        a1 = a1_ref[...]
        inv = 1.0 / (a0[:, F:F + 1] + a1[:, F:F + 1] + 1.0)
        mean = (a0[:, :F] + a1[:, :F] + xv) * inv
        h = jnp.dot(xv, ws_ref[...], preferred_element_type=jnp.float32)
        h = h + jnp.dot(mean, wn_ref[...], preferred_element_type=jnp.float32)
        h = h + b_ref[...]
        o_ref[...] = jnp.maximum(h, 0.0)
        invd_ref[...] = inv

    return pl.pallas_call(
        body,
        grid=grid,
        in_specs=[
            pl.BlockSpec((rblk, F), lambda i: (i, 0)),
            pl.BlockSpec((rblk, w1), lambda i: (i, 0)),
            pl.BlockSpec((rblk, w1), lambda i: (i, 0)),
            pl.BlockSpec((F, F), lambda i: (0, 0)),
            pl.BlockSpec((F, F), lambda i: (0, 0)),
            pl.BlockSpec((1, F), lambda i: (0, 0)),
        ],
        out_specs=[
            pl.BlockSpec((rblk, F), lambda i: (i, 0)),
            pl.BlockSpec((rblk, 1), lambda i: (i, 0)),
        ],
        out_shape=[
            jax.ShapeDtypeStruct((n, F), jnp.float32),
            jax.ShapeDtypeStruct((n, 1), jnp.float32),
        ],
    )(x, p0, p1, w_self, w_neigh, b.reshape(1, F))


def _tc_layer2(h, q0, q1, invd, w_self, w_neigh, b):
    n = h.shape[0]
    rblk = 1000
    grid = (n // rblk,)

    def body(h_ref, a0_ref, a1_ref, invd_ref, ws_ref, wn_ref, b_ref, o_ref):
        hv = h_ref[...]
        mean = (a0_ref[...] + a1_ref[...] + hv) * invd_ref[...]
        o = jnp.dot(hv, ws_ref[...], preferred_element_type=jnp.float32)
        o = o + jnp.dot(mean, wn_ref[...], preferred_element_type=jnp.float32)
        o_ref[...] = o + b_ref[...]

    return pl.pallas_call(
        body,
        grid=grid,
        in_specs=[
            pl.BlockSpec((rblk, F), lambda i: (i, 0)),
            pl.BlockSpec((rblk, F), lambda i: (i, 0)),
            pl.BlockSpec((rblk, F), lambda i: (i, 0)),
            pl.BlockSpec((rblk, 1), lambda i: (i, 0)),
            pl.BlockSpec((F, F), lambda i: (0, 0)),
            pl.BlockSpec((F, F), lambda i: (0, 0)),
            pl.BlockSpec((1, F), lambda i: (0, 0)),
        ],
        out_specs=pl.BlockSpec((rblk, F), lambda i: (i, 0)),
        out_shape=jax.ShapeDtypeStruct((n, F), jnp.float32),
    )(h, q0, q1, invd, w_self, w_neigh, b.reshape(1, F))


def kernel(x, edge_index, W_self1, W_neigh1, b1, W_self2, W_neigh2, b2):
    n = x.shape[0]
    src = edge_index[0].astype(jnp.int32)
    dst = edge_index[1].astype(jnp.int32)
    e = src.shape[0]

    rows_per_tile = (-(-(n + 1) // 16) + 7) // 8 * 8
    n_pad = rows_per_tile * 16

    ntiles = 32
    ntail = 5
    ept = -(-e // ntiles)
    cpt = 2 + 2 * NROW * pl.cdiv(pl.cdiv(ept, CHUNK) - 2, 2 * NROW)
    padpt = cpt * CHUNK - ept
    gpad = ntiles * ept - e
    src_t = jnp.concatenate(
        [src, jnp.full((gpad,), n, jnp.int32)]).reshape(ntiles, ept)
    dst_t = jnp.concatenate(
        [dst, jnp.full((gpad,), n, jnp.int32)]).reshape(ntiles, ept)
    npad1 = padpt + ntail * CHUNK
    junk = (n + (jnp.arange(npad1, dtype=jnp.int32) % (n_pad - n)))
    src_t = jnp.concatenate(
        [src_t, jnp.full((ntiles, npad1), n, jnp.int32)], axis=1)
    dst_t = jnp.concatenate(
        [dst_t, jnp.broadcast_to(junk, (ntiles, npad1))], axis=1)
    idx3 = jnp.stack([src_t.reshape(ntiles, cpt + ntail, CHUNK),
                      dst_t.reshape(ntiles, cpt + ntail, CHUNK)], axis=2)

    w1 = F + 16
    xt = jnp.concatenate(
        [x, jnp.ones((n, 1), x.dtype), jnp.zeros((n, w1 - F - 1), x.dtype)], axis=1)
    xt = jnp.concatenate([xt, jnp.zeros((1, w1), x.dtype)], axis=0)

    p = _sc_segment_sum(xt, idx3, jnp.zeros((n_pad, w1), jnp.float32),
                        w1, cpt)
    h, invd = _tc_layer1(x, p[0, :n], p[1, :n], W_self1, W_neigh1, b1)

    ht = jnp.concatenate([h, jnp.zeros((1, F), h.dtype)], axis=0)
    q = _sc_segment_sum(ht, idx3, jnp.zeros((n_pad, F), jnp.float32),
                        F, cpt)
    return _tc_layer2(h, q[0, :n], q[1, :n], invd, W_self2, W_neigh2, b2)


# speedup vs baseline: 1.0557x; 1.0557x over previous
import functools

import jax
import jax.numpy as jnp
from jax import lax
from jax.experimental import pallas as pl
from jax.experimental.pallas import tpu as pltpu
from jax.experimental.pallas import tpu_sc as plsc

F = 128
CHUNK = 24
NROW = 4
NIDX = 8


def _sc_segment_sum(table, idx3, zrows, width, cpt):
    nrow = table.shape[0]
    n = nrow - 1
    info = plsc.get_sparse_core_info()
    ncores, nsub = info.num_cores, info.num_subcores
    n_pad = zrows.shape[0]
    rows_per_tile = n_pad // nsub
    assert n_pad >= nrow and rows_per_tile % 8 == 0 and (cpt - 2) % (2 * NROW) == 0

    mesh = plsc.VectorSubcoreMesh(core_axis_name="c", subcore_axis_name="s")

    @functools.partial(
        pl.kernel,
        mesh=mesh,
        out_type=jax.ShapeDtypeStruct((ncores, n_pad, width), jnp.float32),
        scratch_types=(
            [pltpu.VMEM((2, CHUNK), jnp.int32) for _ in range(NIDX)]
            + [pltpu.VMEM((CHUNK, width), jnp.float32) for _ in range(NROW)]
            + [pltpu.VMEM_SHARED((n_pad, width), jnp.float32)]
            + [pltpu.SemaphoreType.DMA for _ in range(NIDX + 2 * NROW + 1)]
        ),
        compiler_params=pltpu.CompilerParams(use_tc_tiling_on_sc=False),
    )
    def agg(table_hbm, idx_hbm, z_hbm, out_hbm, *rest):
        idxb = rest[:NIDX]
        rows = rest[NIDX:NIDX + NROW]
        acc = rest[NIDX + NROW]
        sems = rest[NIDX + NROW + 1:]
        isem = sems[:NIDX]
        gsem = sems[NIDX:NIDX + NROW]
        ssem = sems[NIDX + NROW:NIDX + 2 * NROW]
        zsem = sems[NIDX + 2 * NROW]

        c = lax.axis_index("c")
        s = lax.axis_index("s")
        tile = c * nsub + s
        row0 = s * rows_per_tile

        zero_cp = pltpu.make_async_copy(
            z_hbm.at[pl.ds(row0, rows_per_tile)],
            acc.at[pl.ds(row0, rows_per_tile)], zsem)
        zero_cp.start()

        def idx_load(i, q):
            return pltpu.make_async_copy(idx_hbm.at[tile, i], idxb[q], isem[q])

        def gather(q, r):
            return pltpu.make_async_copy(
                table_hbm.at[idxb[q].at[0]], rows[r], gsem[r])

        class _Scatter:
            def __init__(self, q, r):
                self.q, self.r = q, r

            def start(self):
                pltpu.async_copy(rows[self.r], acc.at[idxb[self.q].at[1]],
                                 ssem[self.r], add=True)

            def wait(self):
                pltpu.make_async_copy(rows[self.r],
                                      acc.at[idxb[self.q].at[1]],
                                      ssem[self.r]).wait()

        def scatter(q, r):
            return _Scatter(q, r)

        for j in range(5):
            idx_load(j, j).start()
        for j in range(2):
            idx_load(j, j).wait()
            gather(j, j).start()

        zero_cp.wait()
        plsc.subcore_barrier()

        for i in range(2):
            idx_load(i + 2, i + 2).wait()
            gather(i + 2, (i + 2) % NROW).start()
            idx_load(i + 5, (i + 5) % NIDX).start()
            gather(i, i).wait()
            scatter(i, i).start()

        def rnd(rr, _):
            i0 = rr * (2 * NROW) + 2
            for b in range(2 * NROW):
                i = i0 + b
                scatter(0, b % NROW).wait()
                idx_load(i + 2, (4 + b) % NIDX).wait()
                gather((4 + b) % NIDX, b % NROW).start()
                idx_load(i + 5, (7 + b) % NIDX).start()
                gather(0, (2 + b) % NROW).wait()
                scatter((2 + b) % NIDX, (2 + b) % NROW).start()
            return 0
        lax.fori_loop(0, (cpt - 2) // (2 * NROW), rnd, 0)

        for i in range(cpt - 2, cpt):
            scatter(0, i % NROW).wait()
        for i in range(cpt, cpt + 2):
            gather(0, i % NROW).wait()
        for i in range(cpt + 2, cpt + 5):
            idx_load(i, i % NIDX).wait()
        plsc.subcore_barrier()

        pltpu.sync_copy(acc.at[pl.ds(row0, rows_per_tile)],
                        out_hbm.at[c, pl.ds(row0, rows_per_tile)])

    return agg(table, idx3, zrows)


def _tc_layer1(x, p0, p1, w_self, w_neigh, b):
    n = x.shape[0]
    w1 = p0.shape[1]
    rblk = 1000
    grid = (n // rblk,)

    def body(x_ref, a0_ref, a1_ref, ws_ref, wn_ref, b_ref, o_ref, invd_ref):
        xv = x_ref[...]
        a0 = a0_ref[...]
        a1 = a1_ref[...]
        inv = 1.0 / (a0[:, F:F + 1] + a1[:, F:F + 1] + 1.0)
        mean = (a0[:, :F] + a1[:, :F] + xv) * inv
        h = jnp.dot(xv, ws_ref[...], preferred_element_type=jnp.float32)
        h = h + jnp.dot(mean, wn_ref[...], preferred_element_type=jnp.float32)
        h = h + b_ref[...]
        o_ref[...] = jnp.maximum(h, 0.0)
        invd_ref[...] = inv

    return pl.pallas_call(
        body,
        grid=grid,
        in_specs=[
            pl.BlockSpec((rblk, F), lambda i: (i, 0)),
            pl.BlockSpec((rblk, w1), lambda i: (i, 0)),
            pl.BlockSpec((rblk, w1), lambda i: (i, 0)),
            pl.BlockSpec((F, F), lambda i: (0, 0)),
            pl.BlockSpec((F, F), lambda i: (0, 0)),
            pl.BlockSpec((1, F), lambda i: (0, 0)),
        ],
        out_specs=[
            pl.BlockSpec((rblk, F), lambda i: (i, 0)),
            pl.BlockSpec((rblk, 1), lambda i: (i, 0)),
        ],
        out_shape=[
            jax.ShapeDtypeStruct((n, F), jnp.float32),
            jax.ShapeDtypeStruct((n, 1), jnp.float32),
        ],
    )(x, p0, p1, w_self, w_neigh, b.reshape(1, F))


def _tc_layer2(h, q0, q1, invd, w_self, w_neigh, b):
    n = h.shape[0]
    rblk = 1000
    grid = (n // rblk,)

    def body(h_ref, a0_ref, a1_ref, invd_ref, ws_ref, wn_ref, b_ref, o_ref):
        hv = h_ref[...]
        mean = (a0_ref[...] + a1_ref[...] + hv) * invd_ref[...]
        o = jnp.dot(hv, ws_ref[...], preferred_element_type=jnp.float32)
        o = o + jnp.dot(mean, wn_ref[...], preferred_element_type=jnp.float32)
        o_ref[...] = o + b_ref[...]

    return pl.pallas_call(
        body,
        grid=grid,
        in_specs=[
            pl.BlockSpec((rblk, F), lambda i: (i, 0)),
            pl.BlockSpec((rblk, F), lambda i: (i, 0)),
            pl.BlockSpec((rblk, F), lambda i: (i, 0)),
            pl.BlockSpec((rblk, 1), lambda i: (i, 0)),
            pl.BlockSpec((F, F), lambda i: (0, 0)),
            pl.BlockSpec((F, F), lambda i: (0, 0)),
            pl.BlockSpec((1, F), lambda i: (0, 0)),
        ],
        out_specs=pl.BlockSpec((rblk, F), lambda i: (i, 0)),
        out_shape=jax.ShapeDtypeStruct((n, F), jnp.float32),
    )(h, q0, q1, invd, w_self, w_neigh, b.reshape(1, F))


def kernel(x, edge_index, W_self1, W_neigh1, b1, W_self2, W_neigh2, b2):
    n = x.shape[0]
    src = edge_index[0].astype(jnp.int32)
    dst = edge_index[1].astype(jnp.int32)
    e = src.shape[0]

    rows_per_tile = (-(-(n + 1) // 16) + 7) // 8 * 8
    n_pad = rows_per_tile * 16

    ntiles = 32
    ntail = 5
    ept = -(-e // ntiles)
    cpt = 2 + 2 * NROW * pl.cdiv(pl.cdiv(ept, CHUNK) - 2, 2 * NROW)
    padpt = cpt * CHUNK - ept
    gpad = ntiles * ept - e
    src_t = jnp.concatenate(
        [src, jnp.full((gpad,), n, jnp.int32)]).reshape(ntiles, ept)
    dst_t = jnp.concatenate(
        [dst, jnp.full((gpad,), n, jnp.int32)]).reshape(ntiles, ept)
    npad1 = padpt + ntail * CHUNK
    junk = (n + (jnp.arange(npad1, dtype=jnp.int32) % (n_pad - n)))
    src_t = jnp.concatenate(
        [src_t, jnp.full((ntiles, npad1), n, jnp.int32)], axis=1)
    dst_t = jnp.concatenate(
        [dst_t, jnp.broadcast_to(junk, (ntiles, npad1))], axis=1)
    idx3 = jnp.stack([src_t.reshape(ntiles, cpt + ntail, CHUNK),
                      dst_t.reshape(ntiles, cpt + ntail, CHUNK)], axis=2)

    w1 = F + 16
    xt = jnp.concatenate(
        [x, jnp.ones((n, 1), x.dtype), jnp.zeros((n, w1 - F - 1), x.dtype)], axis=1)
    xt = jnp.concatenate([xt, jnp.zeros((1, w1), x.dtype)], axis=0)

    p = _sc_segment_sum(xt, idx3, jnp.zeros((n_pad, w1), jnp.float32),
                        w1, cpt)
    h, invd = _tc_layer1(x, p[0, :n], p[1, :n], W_self1, W_neigh1, b1)

    ht = jnp.concatenate([h, jnp.zeros((1, F), h.dtype)], axis=0)
    q = _sc_segment_sum(ht, idx3, jnp.zeros((n_pad, F), jnp.float32),
                        F, cpt)
    return _tc_layer2(h, q[0, :n], q[1, :n], invd, W_self2, W_neigh2, b2)
